# Initial kernel scaffold; baseline (speedup 1.0000x reference)
import jax
import jax.numpy as jnp
from jax.experimental import pallas as pl


def kernel(sr_data, tg_data, sr_rel_data, tg_rel_data, edge_index_sr, edge_index_tg, ent_emb_sr, ent_emb_tg, rel_emb_sr, rel_emb_tg, W1, W2):
    raise NotImplementedError("write your pallas kernel here")


# speedup vs baseline: 9.0743x; 9.0743x over previous
import functools

import jax
import jax.numpy as jnp
from jax import lax
from jax.experimental import pallas as pl
from jax.experimental.pallas import tpu as pltpu
from jax.experimental.pallas import tpu_sc as plsc

N = 10000
D = 128
R = 1000
B = 16384
NN = 10112
NC = 2
NS = 16
CH = 128
L = 16

_MESH = plsc.VectorSubcoreMesh(core_axis_name="c", subcore_axis_name="s")


def _pad_to(arr, m, fill):
    pad = (-arr.shape[0]) % m
    if pad == 0:
        return arr
    return jnp.concatenate([arr, jnp.full((pad,), fill, dtype=arr.dtype)])


def _sc_degree_hist(hist_idx):
    he = hist_idx.shape[0] // NC
    per_w = he // NS
    n_chunks = per_w // CH
    rpt = (2 * NN) // NS

    @functools.partial(
        pl.kernel,
        out_type=jax.ShapeDtypeStruct((NC * 2 * NN,), jnp.float32),
        mesh=_MESH,
        scratch_types=[
            pltpu.VMEM((CH,), jnp.int32),
            pltpu.VMEM((CH,), jnp.float32),
            pltpu.VMEM((rpt,), jnp.float32),
            pltpu.VMEM_SHARED((2 * NN,), jnp.float32),
        ],
    )
    def k(idx_hbm, out_hbm, idxv, onesv, outv, acc):
        cid = lax.axis_index("c")
        sid = lax.axis_index("s")
        r0 = sid * rpt
        ones16 = jnp.ones((L,), jnp.float32)
        zeros16 = jnp.zeros((L,), jnp.float32)
        for r in range(CH // L):
            onesv[pl.ds(r * L, L)] = ones16
        for r in range(rpt // L):
            outv[pl.ds(r * L, L)] = zeros16
        pltpu.sync_copy(outv, acc.at[pl.ds(r0, rpt)])
        plsc.subcore_barrier()

        base_w = cid * he + sid * per_w

        @pl.loop(0, n_chunks)
        def _(i):
            pltpu.sync_copy(idx_hbm.at[pl.ds(base_w + i * CH, CH)], idxv)
            pltpu.sync_copy(onesv, acc.at[idxv], add=True)

        plsc.subcore_barrier()
        pltpu.sync_copy(acc.at[pl.ds(r0, rpt)], outv)
        pltpu.sync_copy(outv, out_hbm.at[pl.ds(cid * 2 * NN + r0, rpt)])

    return k(hist_idx)


def _sc_propagate(h_all, src_all, dst_all, zeros_chunk):
    ep = src_all.shape[0] // NC
    per_w = ep // NS
    n_chunks = per_w // CH
    rpt = NN // NS

    @functools.partial(
        pl.kernel,
        out_type=jax.ShapeDtypeStruct((NC * NN, D), jnp.float32),
        mesh=_MESH,
        scratch_types=[
            pltpu.VMEM((CH,), jnp.int32),
            pltpu.VMEM((CH,), jnp.int32),
            pltpu.VMEM((CH, D), jnp.float32),
            pltpu.VMEM((CH, D), jnp.float32),
            pltpu.VMEM_SHARED((NN, D), jnp.float32),
            pltpu.SemaphoreType.DMA,
        ],
    )
    def k(h_hbm, src_hbm, dst_hbm, zero_hbm, out_hbm,
          sidx, didx, rows, stage, acc, sem):
        cid = lax.axis_index("c")
        sid = lax.axis_index("s")
        r0 = sid * rpt
        pltpu.sync_copy(zero_hbm, stage)
        for off in range(0, rpt, CH):
            c = min(CH, rpt - off)
            pltpu.sync_copy(stage.at[pl.ds(0, c)], acc.at[pl.ds(r0 + off, c)])
        plsc.subcore_barrier()

        base_w = cid * ep + sid * per_w

        @pl.loop(0, n_chunks)
        def _(i):
            base = base_w + i * CH
            pltpu.sync_copy(src_hbm.at[pl.ds(base, CH)], sidx)
            pltpu.sync_copy(dst_hbm.at[pl.ds(base, CH)], didx)
            pltpu.async_copy(h_hbm.at[sidx], rows, sem).wait()
            pltpu.sync_copy(rows, acc.at[didx], add=True)

        plsc.subcore_barrier()
        for off in range(0, rpt, CH):
            c = min(CH, rpt - off)
            pltpu.sync_copy(acc.at[pl.ds(r0 + off, c)], stage.at[pl.ds(0, c)])
            pltpu.sync_copy(stage.at[pl.ds(0, c)],
                            out_hbm.at[pl.ds(cid * NN + r0 + off, c)])

    return k(h_all, src_all, dst_all, zeros_chunk)


def _sc_gather(table, idx):
    per_w = B // (NC * NS)
    n_chunks = per_w // CH

    @functools.partial(
        pl.kernel,
        out_type=jax.ShapeDtypeStruct((B, D), jnp.float32),
        mesh=_MESH,
        scratch_types=[
            pltpu.VMEM((CH,), jnp.int32),
            pltpu.VMEM((CH, D), jnp.float32),
            pltpu.SemaphoreType.DMA,
        ],
    )
    def k(table_hbm, idx_hbm, out_hbm, idxv, rows, sem):
        cid = lax.axis_index("c")
        sid = lax.axis_index("s")
        wid = sid * NC + cid
        for i in range(n_chunks):
            base = wid * per_w + i * CH
            pltpu.sync_copy(idx_hbm.at[pl.ds(base, CH)], idxv)
            pltpu.async_copy(table_hbm.at[idxv], rows, sem).wait()
            pltpu.sync_copy(rows, out_hbm.at[pl.ds(base, CH)])

    return k(table, idx)


_BLK = 128


def _rs(d_ref):
    return lax.rsqrt(jnp.maximum(d_ref[...], 1.0))


def _tc_scale_matmul(x, deg_o, w):
    m = x.shape[0]

    def body(x_ref, d_ref, w_ref, o_ref):
        o_ref[...] = jnp.dot(x_ref[...] * _rs(d_ref), w_ref[...],
                             preferred_element_type=jnp.float32)

    return pl.pallas_call(
        body,
        grid=(m // _BLK,),
        in_specs=[
            pl.BlockSpec((_BLK, D), lambda i: (i, 0)),
            pl.BlockSpec((_BLK, 1), lambda i: (i, 0)),
            pl.BlockSpec((D, D), lambda i: (0, 0)),
        ],
        out_specs=pl.BlockSpec((_BLK, D), lambda i: (i, 0)),
        out_shape=jax.ShapeDtypeStruct((m, D), jnp.float32),
    )(x, deg_o, w)


def _tc_combine_matmul(a, deg_i, deg_o, w):
    m = a.shape[0]

    def body(a_ref, di_ref, do_ref, w_ref, o_ref):
        g = jnp.maximum(a_ref[...] * _rs(di_ref), 0.0)
        o_ref[...] = jnp.dot(g * _rs(do_ref), w_ref[...],
                             preferred_element_type=jnp.float32)

    return pl.pallas_call(
        body,
        grid=(m // _BLK,),
        in_specs=[
            pl.BlockSpec((_BLK, D), lambda i: (i, 0)),
            pl.BlockSpec((_BLK, 1), lambda i: (i, 0)),
            pl.BlockSpec((_BLK, 1), lambda i: (i, 0)),
            pl.BlockSpec((D, D), lambda i: (0, 0)),
        ],
        out_specs=pl.BlockSpec((_BLK, D), lambda i: (i, 0)),
        out_shape=jax.ShapeDtypeStruct((m, D), jnp.float32),
    )(a, deg_i, deg_o, w)


def _tc_combine(a, deg_i):
    m = a.shape[0]

    def body(a_ref, di_ref, o_ref):
        o_ref[...] = jnp.maximum(a_ref[...] * _rs(di_ref), 0.0)

    return pl.pallas_call(
        body,
        grid=(m // _BLK,),
        in_specs=[
            pl.BlockSpec((_BLK, D), lambda i: (i, 0)),
            pl.BlockSpec((_BLK, 1), lambda i: (i, 0)),
        ],
        out_specs=pl.BlockSpec((_BLK, D), lambda i: (i, 0)),
        out_shape=jax.ShapeDtypeStruct((m, D), jnp.float32),
    )(a, deg_i)


def kernel(sr_data, tg_data, sr_rel_data, tg_rel_data,
           edge_index_sr, edge_index_tg,
           ent_emb_sr, ent_emb_tg, rel_emb_sr, rel_emb_tg, W1, W2):
    i32 = jnp.int32
    src_sr = edge_index_sr[0].astype(i32)
    dst_sr = edge_index_sr[1].astype(i32)
    src_tg = edge_index_tg[0].astype(i32)
    dst_tg = edge_index_tg[1].astype(i32)

    half_m = NS * CH
    hist_sr = _pad_to(jnp.concatenate([src_sr, dst_sr + NN]), half_m, N)
    hist_tg = _pad_to(jnp.concatenate([src_tg, dst_tg + NN]), half_m, N)
    deg = _sc_degree_hist(jnp.concatenate([hist_sr, hist_tg]))
    deg_o = jnp.concatenate([deg[0:NN], deg[2 * NN:3 * NN]])[:, None]
    deg_i = jnp.concatenate([deg[NN:2 * NN], deg[3 * NN:4 * NN]])[:, None]

    epm = NS * CH
    src_all = jnp.concatenate([_pad_to(src_sr, epm, 0),
                               _pad_to(src_tg, epm, 0) + NN])
    dst_all = jnp.concatenate([_pad_to(dst_sr, epm, N),
                               _pad_to(dst_tg, epm, N)])
    zeros_chunk = jnp.zeros((CH, D), jnp.float32)

    zpad = jnp.zeros((NN - N, D), jnp.float32)
    x_all = jnp.concatenate([ent_emb_sr, zpad, ent_emb_tg, zpad])

    h1 = _tc_scale_matmul(x_all, deg_o, W1)
    acc1 = _sc_propagate(h1, src_all, dst_all, zeros_chunk)
    h2 = _tc_combine_matmul(acc1, deg_i, deg_o, W2)
    acc2 = _sc_propagate(h2, src_all, dst_all, zeros_chunk)
    g = _tc_combine(acc2, deg_i)

    return (_sc_gather(g, sr_data.astype(i32)),
            _sc_gather(g, tg_data.astype(i32) + NN),
            _sc_gather(rel_emb_sr, sr_rel_data.astype(i32)),
            _sc_gather(rel_emb_tg, tg_rel_data.astype(i32)))


# speedup vs baseline: 12.7907x; 1.4096x over previous
import functools

import jax
import jax.numpy as jnp
from jax import lax
from jax.experimental import pallas as pl
from jax.experimental.pallas import tpu as pltpu
from jax.experimental.pallas import tpu_sc as plsc

N = 10000
D = 128
R = 1000
B = 16384
NN = 10112
NC = 2
NS = 16
CH = 128
L = 16

_MESH = plsc.VectorSubcoreMesh(core_axis_name="c", subcore_axis_name="s")


def _pad_to(arr, m, fill):
    pad = (-arr.shape[0]) % m
    if pad == 0:
        return arr
    return jnp.concatenate([arr, jnp.full((pad,), fill, dtype=arr.dtype)])


def _sc_degree_hist(hist_idx):
    he = hist_idx.shape[0] // NC
    per_w = he // NS
    n_chunks = per_w // CH
    rpt = (2 * NN) // NS

    @functools.partial(
        pl.kernel,
        out_type=jax.ShapeDtypeStruct((NC * 2 * NN,), jnp.float32),
        mesh=_MESH,
        scratch_types=[
            pltpu.VMEM((CH,), jnp.int32),
            pltpu.VMEM((CH,), jnp.float32),
            pltpu.VMEM((rpt,), jnp.float32),
            pltpu.VMEM_SHARED((2 * NN,), jnp.float32),
        ],
    )
    def k(idx_hbm, out_hbm, idxv, onesv, outv, acc):
        cid = lax.axis_index("c")
        sid = lax.axis_index("s")
        r0 = sid * rpt
        ones16 = jnp.ones((L,), jnp.float32)
        zeros16 = jnp.zeros((L,), jnp.float32)
        for r in range(CH // L):
            onesv[pl.ds(r * L, L)] = ones16
        for r in range(rpt // L):
            outv[pl.ds(r * L, L)] = zeros16
        pltpu.sync_copy(outv, acc.at[pl.ds(r0, rpt)])
        plsc.subcore_barrier()

        base_w = cid * he + sid * per_w

        @pl.loop(0, n_chunks)
        def _(i):
            pltpu.sync_copy(idx_hbm.at[pl.ds(base_w + i * CH, CH)], idxv)
            pltpu.sync_copy(onesv, acc.at[idxv], add=True)

        plsc.subcore_barrier()
        pltpu.sync_copy(acc.at[pl.ds(r0, rpt)], outv)
        pltpu.sync_copy(outv, out_hbm.at[pl.ds(cid * 2 * NN + r0, rpt)])

    return k(hist_idx)


_NB = 2


def _sc_propagate(h_all, src_all, dst_all, zeros_chunk):
    ep = src_all.shape[0] // NC
    per_w = ep // NS
    n_chunks = per_w // CH
    rpt = NN // NS

    @functools.partial(
        pl.kernel,
        out_type=jax.ShapeDtypeStruct((NC * NN, D), jnp.float32),
        mesh=_MESH,
        scratch_types=(
            [pltpu.VMEM((CH,), jnp.int32)] * (2 * _NB)
            + [pltpu.VMEM((CH, D), jnp.float32)] * _NB
            + [pltpu.VMEM_SHARED((NN, D), jnp.float32)]
            + [pltpu.SemaphoreType.DMA] * _NB
        ),
    )
    def k(h_hbm, src_hbm, dst_hbm, zero_hbm, out_hbm,
          si0, si1, di0, di1, rw0, rw1,
          acc, gs0, gs1):
        sidx = (si0, si1)
        didx = (di0, di1)
        rows = (rw0, rw1)
        gsem = (gs0, gs1)
        cid = lax.axis_index("c")
        sid = lax.axis_index("s")
        r0 = sid * rpt
        pltpu.sync_copy(zero_hbm, rows[0])
        for off in range(0, rpt, CH):
            c = min(CH, rpt - off)
            pltpu.sync_copy(rows[0].at[pl.ds(0, c)],
                            acc.at[pl.ds(r0 + off, c)])
        plsc.subcore_barrier()

        base_w = cid * ep + sid * per_w

        def load_and_fire(b, chunk):
            base = base_w + chunk * CH
            pltpu.sync_copy(src_hbm.at[pl.ds(base, CH)], sidx[b])
            pltpu.sync_copy(dst_hbm.at[pl.ds(base, CH)], didx[b])
            pltpu.async_copy(h_hbm.at[sidx[b]], rows[b], gsem[b])

        for b in range(_NB):
            load_and_fire(b, b)

        @pl.loop(0, n_chunks // _NB)
        def _(j):
            for b in range(_NB):
                c3 = j * _NB + b
                pltpu.make_async_copy(h_hbm.at[sidx[b]], rows[b],
                                      gsem[b]).wait()
                pltpu.sync_copy(rows[b], acc.at[didx[b]], add=True)

                @pl.when(c3 + _NB < n_chunks)
                def _():
                    load_and_fire(b, c3 + _NB)

        plsc.subcore_barrier()
        for off in range(0, rpt, CH):
            c = min(CH, rpt - off)
            pltpu.sync_copy(acc.at[pl.ds(r0 + off, c)],
                            rows[0].at[pl.ds(0, c)])
            pltpu.sync_copy(rows[0].at[pl.ds(0, c)],
                            out_hbm.at[pl.ds(cid * NN + r0 + off, c)])

    return k(h_all, src_all, dst_all, zeros_chunk)


def _sc_gather(table, idx):
    per_w = B // (NC * NS)
    n_chunks = per_w // CH

    @functools.partial(
        pl.kernel,
        out_type=jax.ShapeDtypeStruct((B, D), jnp.float32),
        mesh=_MESH,
        scratch_types=[
            pltpu.VMEM((CH,), jnp.int32),
            pltpu.VMEM((CH, D), jnp.float32),
            pltpu.SemaphoreType.DMA,
        ],
    )
    def k(table_hbm, idx_hbm, out_hbm, idxv, rows, sem):
        cid = lax.axis_index("c")
        sid = lax.axis_index("s")
        wid = sid * NC + cid
        for i in range(n_chunks):
            base = wid * per_w + i * CH
            pltpu.sync_copy(idx_hbm.at[pl.ds(base, CH)], idxv)
            pltpu.async_copy(table_hbm.at[idxv], rows, sem).wait()
            pltpu.sync_copy(rows, out_hbm.at[pl.ds(base, CH)])

    return k(table, idx)


_BLK = 128


def _rs(d_ref):
    return lax.rsqrt(jnp.maximum(d_ref[...], 1.0))


def _tc_scale_matmul(x, deg_o, w):
    m = x.shape[0]

    def body(x_ref, d_ref, w_ref, o_ref):
        o_ref[...] = jnp.dot(x_ref[...] * _rs(d_ref), w_ref[...],
                             preferred_element_type=jnp.float32)

    return pl.pallas_call(
        body,
        grid=(m // _BLK,),
        in_specs=[
            pl.BlockSpec((_BLK, D), lambda i: (i, 0)),
            pl.BlockSpec((_BLK, 1), lambda i: (i, 0)),
            pl.BlockSpec((D, D), lambda i: (0, 0)),
        ],
        out_specs=pl.BlockSpec((_BLK, D), lambda i: (i, 0)),
        out_shape=jax.ShapeDtypeStruct((m, D), jnp.float32),
    )(x, deg_o, w)


def _tc_combine_matmul(a, deg_i, deg_o, w):
    m = a.shape[0]

    def body(a_ref, di_ref, do_ref, w_ref, o_ref):
        g = jnp.maximum(a_ref[...] * _rs(di_ref), 0.0)
        o_ref[...] = jnp.dot(g * _rs(do_ref), w_ref[...],
                             preferred_element_type=jnp.float32)

    return pl.pallas_call(
        body,
        grid=(m // _BLK,),
        in_specs=[
            pl.BlockSpec((_BLK, D), lambda i: (i, 0)),
            pl.BlockSpec((_BLK, 1), lambda i: (i, 0)),
            pl.BlockSpec((_BLK, 1), lambda i: (i, 0)),
            pl.BlockSpec((D, D), lambda i: (0, 0)),
        ],
        out_specs=pl.BlockSpec((_BLK, D), lambda i: (i, 0)),
        out_shape=jax.ShapeDtypeStruct((m, D), jnp.float32),
    )(a, deg_i, deg_o, w)


def _tc_combine(a, deg_i):
    m = a.shape[0]

    def body(a_ref, di_ref, o_ref):
        o_ref[...] = jnp.maximum(a_ref[...] * _rs(di_ref), 0.0)

    return pl.pallas_call(
        body,
        grid=(m // _BLK,),
        in_specs=[
            pl.BlockSpec((_BLK, D), lambda i: (i, 0)),
            pl.BlockSpec((_BLK, 1), lambda i: (i, 0)),
        ],
        out_specs=pl.BlockSpec((_BLK, D), lambda i: (i, 0)),
        out_shape=jax.ShapeDtypeStruct((m, D), jnp.float32),
    )(a, deg_i)


def kernel(sr_data, tg_data, sr_rel_data, tg_rel_data,
           edge_index_sr, edge_index_tg,
           ent_emb_sr, ent_emb_tg, rel_emb_sr, rel_emb_tg, W1, W2):
    i32 = jnp.int32
    src_sr = edge_index_sr[0].astype(i32)
    dst_sr = edge_index_sr[1].astype(i32)
    src_tg = edge_index_tg[0].astype(i32)
    dst_tg = edge_index_tg[1].astype(i32)

    half_m = NS * CH
    hist_sr = _pad_to(jnp.concatenate([src_sr, dst_sr + NN]), half_m, N)
    hist_tg = _pad_to(jnp.concatenate([src_tg, dst_tg + NN]), half_m, N)
    deg = _sc_degree_hist(jnp.concatenate([hist_sr, hist_tg]))
    deg_o = jnp.concatenate([deg[0:NN], deg[2 * NN:3 * NN]])[:, None]
    deg_i = jnp.concatenate([deg[NN:2 * NN], deg[3 * NN:4 * NN]])[:, None]

    epm = NS * CH * _NB

    def _pad_spread(arr, lo, hi):
        pad = (-arr.shape[0]) % epm
        fill = lo + jnp.arange(pad, dtype=arr.dtype) % (hi - lo)
        return jnp.concatenate([arr, fill])

    src_all = jnp.concatenate([_pad_spread(src_sr, 0, N),
                               _pad_spread(src_tg, 0, N) + NN])
    dst_all = jnp.concatenate([_pad_spread(dst_sr, N, NN),
                               _pad_spread(dst_tg, N, NN)])
    zeros_chunk = jnp.zeros((CH, D), jnp.float32)

    zpad = jnp.zeros((NN - N, D), jnp.float32)
    x_all = jnp.concatenate([ent_emb_sr, zpad, ent_emb_tg, zpad])

    h1 = _tc_scale_matmul(x_all, deg_o, W1)
    acc1 = _sc_propagate(h1, src_all, dst_all, zeros_chunk)
    h2 = _tc_combine_matmul(acc1, deg_i, deg_o, W2)
    acc2 = _sc_propagate(h2, src_all, dst_all, zeros_chunk)
    g = _tc_combine(acc2, deg_i)

    return (_sc_gather(g, sr_data.astype(i32)),
            _sc_gather(g, tg_data.astype(i32) + NN),
            _sc_gather(rel_emb_sr, sr_rel_data.astype(i32)),
            _sc_gather(rel_emb_tg, tg_rel_data.astype(i32)))
